# baseline (speedup 1.0000x reference)
import functools

import jax
import jax.numpy as jnp
from jax import lax
from jax.experimental import pallas as pl
from jax.experimental.pallas import tpu as pltpu
from jax.experimental.pallas import tpu_sc as plsc

_NUM_CORES = 2
_NUM_SUBCORES = 16
_CHUNK = 16


def kernel(x, emb_weight):
    seq_len = x.shape[1]
    dim = emb_weight.shape[1]
    num_workers = _NUM_CORES * _NUM_SUBCORES
    rows_per_w = seq_len // num_workers
    n_chunks = rows_per_w // _CHUNK

    mesh = plsc.VectorSubcoreMesh(core_axis_name="c", subcore_axis_name="s")

    @functools.partial(
        pl.kernel,
        mesh=mesh,
        out_type=jax.ShapeDtypeStruct((seq_len, dim), emb_weight.dtype),
        scratch_types=[
            pltpu.VMEM((2, _CHUNK, dim), jnp.float32),
            pltpu.SemaphoreType.DMA((2,)),
            pltpu.SemaphoreType.DMA((2,)),
        ],
    )
    def sc_copy(table_hbm, out_hbm, buf, in_sems, out_sems):
        wid = lax.axis_index("s") * _NUM_CORES + lax.axis_index("c")
        base = wid * rows_per_w

        def in_copy(g):
            return pltpu.make_async_copy(
                table_hbm.at[pl.ds(base + g * _CHUNK, _CHUNK)],
                buf.at[g % 2],
                in_sems.at[g % 2],
            )

        def out_copy(g):
            return pltpu.make_async_copy(
                buf.at[g % 2],
                out_hbm.at[pl.ds(base + g * _CHUNK, _CHUNK)],
                out_sems.at[g % 2],
            )

        in_copy(0).start()
        for g in range(n_chunks):
            if g + 1 < n_chunks:
                if g >= 1:
                    out_copy(g - 1).wait()
                in_copy(g + 1).start()
            in_copy(g).wait()
            out_copy(g).start()
        out_copy(n_chunks - 2).wait()
        out_copy(n_chunks - 1).wait()

    return sc_copy(emb_weight)


# speedup vs baseline: 1.9100x; 1.9100x over previous
import jax
import jax.numpy as jnp
from jax.experimental import pallas as pl
from jax.experimental.pallas import tpu as pltpu


def _copy_kernel(emb_ref, out_ref):
    out_ref[...] = emb_ref[...]


def kernel(x, emb_weight):
    seq_len = x.shape[1]
    dim = emb_weight.shape[1]
    block_rows = 512
    grid = (seq_len // block_rows,)
    return pl.pallas_call(
        _copy_kernel,
        grid=grid,
        in_specs=[pl.BlockSpec((block_rows, dim), lambda i: (i, 0))],
        out_specs=pl.BlockSpec((block_rows, dim), lambda i: (i, 0)),
        out_shape=jax.ShapeDtypeStruct((seq_len, dim), emb_weight.dtype),
        compiler_params=pltpu.CompilerParams(
            dimension_semantics=("parallel",),
        ),
    )(emb_weight)
